# baseline (speedup 1.0000x reference)
import functools

import jax
import jax.numpy as jnp
from jax import lax
from jax.experimental import pallas as pl
from jax.experimental.pallas import tpu as pltpu
from jax.experimental.pallas import tpu_sc as plsc

NC = 2
NS = 16
NW = NC * NS
CHUNK = 128


def _deg_kernel(n_pad, ch, d):
    stripe = n_pad // NS
    mesh = plsc.VectorSubcoreMesh(core_axis_name="c", subcore_axis_name="s",
                                  num_cores=NC, num_subcores=NS)

    @functools.partial(
        pl.kernel,
        out_type=jax.ShapeDtypeStruct((NC * n_pad, d), jnp.float32),
        mesh=mesh,
        scratch_types=[
            pltpu.VMEM((ch, CHUNK), jnp.int32),
            pltpu.VMEM((CHUNK, d), jnp.float32),
            pltpu.VMEM_SHARED((n_pad, d), jnp.float32),
        ],
    )
    def deg_kernel(dst_hbm, ones_hbm, z_hbm, out_hbm, dst_v, ones_v, acc):
        c = lax.axis_index("c")
        s = lax.axis_index("s")
        wid = c * NS + s
        pltpu.sync_copy(dst_hbm.at[wid], dst_v)
        pltpu.sync_copy(ones_hbm, ones_v)
        pltpu.sync_copy(z_hbm.at[pl.ds(s * stripe, stripe)],
                        acc.at[pl.ds(s * stripe, stripe)])
        plsc.subcore_barrier()

        @pl.loop(0, ch)
        def _(j):
            pltpu.sync_copy(ones_v, acc.at[dst_v.at[j]], add=True)

        plsc.subcore_barrier()
        pltpu.sync_copy(acc.at[pl.ds(s * stripe, stripe)],
                        out_hbm.at[pl.ds(c * n_pad + s * stripe, stripe)])

    return deg_kernel


def _layer_kernel(n_pad, ch0, ch1, d):
    stripe = n_pad // NS
    chm = max(ch0, ch1)
    mesh = plsc.VectorSubcoreMesh(core_axis_name="c", subcore_axis_name="s",
                                  num_cores=NC, num_subcores=NS)

    @functools.partial(
        pl.kernel,
        out_type=jax.ShapeDtypeStruct((NC * n_pad, d), jnp.float32),
        mesh=mesh,
        scratch_types=[
            pltpu.VMEM((chm, CHUNK), jnp.int32),
            pltpu.VMEM((chm, CHUNK), jnp.int32),
            pltpu.VMEM((CHUNK, d), jnp.float32),
            pltpu.VMEM_SHARED((n_pad, d), jnp.float32),
        ],
    )
    def layer_kernel(h_hbm, src0_hbm, dst0_hbm, src1_hbm, dst1_hbm,
                     z_hbm, out_hbm, src_v, dst_v, gbuf, acc):
        c = lax.axis_index("c")
        s = lax.axis_index("s")
        pltpu.sync_copy(z_hbm.at[pl.ds(s * stripe, stripe)],
                        acc.at[pl.ds(s * stripe, stripe)])

        @pl.when(c == 0)
        def _():
            pltpu.sync_copy(src0_hbm.at[s], src_v.at[pl.ds(0, ch0)])
            pltpu.sync_copy(dst0_hbm.at[s], dst_v.at[pl.ds(0, ch0)])

        @pl.when(c == 1)
        def _():
            pltpu.sync_copy(src1_hbm.at[s], src_v.at[pl.ds(0, ch1)])
            pltpu.sync_copy(dst1_hbm.at[s], dst_v.at[pl.ds(0, ch1)])

        plsc.subcore_barrier()
        trip = jnp.where(c == 0, ch0, ch1)

        @pl.loop(0, trip)
        def _(j):
            pltpu.sync_copy(h_hbm.at[src_v.at[j]], gbuf)
            pltpu.sync_copy(gbuf, acc.at[dst_v.at[j]], add=True)

        plsc.subcore_barrier()
        pltpu.sync_copy(acc.at[pl.ds(s * stripe, stripe)],
                        out_hbm.at[pl.ds(c * n_pad + s * stripe, stripe)])

    return layer_kernel


def _dis_from_degp(degp, n_pad, n):
    deg = degp[:n_pad, 0:1] + degp[n_pad:, 0:1] + 1.0
    rows = lax.broadcasted_iota(jnp.int32, (n_pad, 1), 0)
    return jnp.where(rows < n, lax.rsqrt(deg), 0.0)


def _prep_body(n_pad, n, x_ref, degp_ref, w_ref, out_ref):
    dis = _dis_from_degp(degp_ref[...], n_pad, n)
    h = jnp.dot(x_ref[...], w_ref[...], preferred_element_type=jnp.float32)
    out_ref[...] = h * dis


def _mid_body(n_pad, n, p_ref, h_ref, degp_ref, b_ref, w_ref, out_ref):
    dis = _dis_from_degp(degp_ref[...], n_pad, n)
    p = p_ref[...]
    acc = p[:n_pad] + p[n_pad:] + h_ref[...]
    t = jnp.maximum(acc * dis + b_ref[...], 0.0)
    out_ref[...] = jnp.dot(
        t, w_ref[...], preferred_element_type=jnp.float32) * dis


def _final_body(n_pad, n, p_ref, h_ref, degp_ref, b_ref, wh_ref, bh_ref,
                emb_ref, pred_ref):
    dis = _dis_from_degp(degp_ref[...], n_pad, n)
    p = p_ref[...]
    acc = p[:n_pad] + p[n_pad:] + h_ref[...]
    emb = jnp.maximum(acc * dis + b_ref[...], 0.0)
    pred = jnp.dot(
        emb, wh_ref[...], preferred_element_type=jnp.float32) + bh_ref[...]
    emb_ref[...] = jnp.nan_to_num(emb)
    pred_ref[...] = jnp.nan_to_num(pred)


def kernel(x, edge_index, W1, b1, W2, b2, W3, b3, Wh, bh):
    x = x.astype(jnp.float32)
    n, d = x.shape
    e = edge_index.shape[1]
    n_pad = ((n + NS * CHUNK - 1) // (NS * CHUNK)) * (NS * CHUNK)
    ch = -(-e // (NW * CHUNK))
    e_pad = NW * ch * CHUNK

    fill = jnp.full((e_pad - e,), n, dtype=edge_index.dtype)
    dst = jnp.concatenate([edge_index[1], fill]).reshape(NW, ch, CHUNK)

    tot = -(-e // (NS * CHUNK))
    ch1 = max(1, int(tot * 0.446))
    ch0 = tot - ch1
    e0 = NS * ch0 * CHUNK
    e1_pad = NS * ch1 * CHUNK
    fill1 = jnp.full((e0 + e1_pad - e,), n, dtype=edge_index.dtype)
    src0 = edge_index[0, :e0].reshape(NS, ch0, CHUNK)
    dst0 = edge_index[1, :e0].reshape(NS, ch0, CHUNK)
    src1 = jnp.concatenate(
        [edge_index[0, e0:], fill1]).reshape(NS, ch1, CHUNK)
    dst1 = jnp.concatenate(
        [edge_index[1, e0:], fill1]).reshape(NS, ch1, CHUNK)

    x_pad = jnp.zeros((n_pad, d), jnp.float32).at[:n].set(x)
    z = jnp.zeros((n_pad, d), jnp.float32)
    ones = jnp.ones((CHUNK, d), jnp.float32)
    b1r = b1.reshape(1, d)
    b2r = b2.reshape(1, d)
    b3r = b3.reshape(1, d)
    bhr = bh.reshape(1, 1)

    deg_k = _deg_kernel(n_pad, ch, d)
    lay_k = _layer_kernel(n_pad, ch0, ch1, d)

    prep = pl.pallas_call(
        functools.partial(_prep_body, n_pad, n),
        out_shape=jax.ShapeDtypeStruct((n_pad, d), jnp.float32))
    mid = pl.pallas_call(
        functools.partial(_mid_body, n_pad, n),
        out_shape=jax.ShapeDtypeStruct((n_pad, d), jnp.float32))
    final = pl.pallas_call(
        functools.partial(_final_body, n_pad, n),
        out_shape=(jax.ShapeDtypeStruct((n_pad, d), jnp.float32),
                   jax.ShapeDtypeStruct((n_pad, 1), jnp.float32)))

    degp = deg_k(dst, ones, z)
    h = prep(x_pad, degp, W1)
    p = lay_k(h, src0, dst0, src1, dst1, z)
    h = mid(p, h, degp, b1r, W2)
    p = lay_k(h, src0, dst0, src1, dst1, z)
    h = mid(p, h, degp, b2r, W3)
    p = lay_k(h, src0, dst0, src1, dst1, z)
    emb, pred = final(p, h, degp, b3r, Wh, bhr)
    return emb[:n], pred[:n, 0]


# speedup vs baseline: 1.0110x; 1.0110x over previous
import functools

import jax
import jax.numpy as jnp
from jax import lax
from jax.experimental import pallas as pl
from jax.experimental.pallas import tpu as pltpu
from jax.experimental.pallas import tpu_sc as plsc

NC = 2
NS = 16
NW = NC * NS
CHUNK = 128


def _deg_kernel(n_pad, ch, d):
    stripe = n_pad // NS
    mesh = plsc.VectorSubcoreMesh(core_axis_name="c", subcore_axis_name="s",
                                  num_cores=NC, num_subcores=NS)

    @functools.partial(
        pl.kernel,
        out_type=jax.ShapeDtypeStruct((NC * n_pad, d), jnp.float32),
        mesh=mesh,
        scratch_types=[
            pltpu.VMEM((ch, CHUNK), jnp.int32),
            pltpu.VMEM((CHUNK, d), jnp.float32),
            pltpu.VMEM_SHARED((n_pad, d), jnp.float32),
        ],
    )
    def deg_kernel(dst_hbm, ones_hbm, z_hbm, out_hbm, dst_v, ones_v, acc):
        c = lax.axis_index("c")
        s = lax.axis_index("s")
        wid = c * NS + s
        pltpu.sync_copy(dst_hbm.at[wid], dst_v)
        pltpu.sync_copy(ones_hbm, ones_v)
        pltpu.sync_copy(z_hbm.at[pl.ds(s * stripe, stripe)],
                        acc.at[pl.ds(s * stripe, stripe)])
        plsc.subcore_barrier()

        @pl.loop(0, ch)
        def _(j):
            pltpu.sync_copy(ones_v, acc.at[dst_v.at[j]], add=True)

        plsc.subcore_barrier()
        pltpu.sync_copy(acc.at[pl.ds(s * stripe, stripe)],
                        out_hbm.at[pl.ds(c * n_pad + s * stripe, stripe)])

    return deg_kernel


def _layer_kernel(n_pad, ch0, ch1, d):
    stripe = n_pad // NS
    chm = max(ch0, ch1)
    mesh = plsc.VectorSubcoreMesh(core_axis_name="c", subcore_axis_name="s",
                                  num_cores=NC, num_subcores=NS)

    @functools.partial(
        pl.kernel,
        out_type=jax.ShapeDtypeStruct((NC * n_pad, d), jnp.float32),
        mesh=mesh,
        scratch_types=[
            pltpu.VMEM((chm, CHUNK), jnp.int32),
            pltpu.VMEM((chm, CHUNK), jnp.int32),
            pltpu.VMEM((CHUNK, d), jnp.float32),
            pltpu.VMEM_SHARED((n_pad, d), jnp.float32),
        ],
    )
    def layer_kernel(h_hbm, src0_hbm, dst0_hbm, src1_hbm, dst1_hbm,
                     z_hbm, out_hbm, src_v, dst_v, gbuf, acc):
        c = lax.axis_index("c")
        s = lax.axis_index("s")
        pltpu.sync_copy(z_hbm.at[pl.ds(s * stripe, stripe)],
                        acc.at[pl.ds(s * stripe, stripe)])

        @pl.when(c == 0)
        def _():
            pltpu.sync_copy(src0_hbm.at[s], src_v.at[pl.ds(0, ch0)])
            pltpu.sync_copy(dst0_hbm.at[s], dst_v.at[pl.ds(0, ch0)])

        @pl.when(c == 1)
        def _():
            pltpu.sync_copy(src1_hbm.at[s], src_v.at[pl.ds(0, ch1)])
            pltpu.sync_copy(dst1_hbm.at[s], dst_v.at[pl.ds(0, ch1)])

        plsc.subcore_barrier()
        trip = jnp.where(c == 0, ch0, ch1)

        @pl.loop(0, trip)
        def _(j):
            pltpu.sync_copy(h_hbm.at[src_v.at[j]], gbuf)
            pltpu.sync_copy(gbuf, acc.at[dst_v.at[j]], add=True)

        plsc.subcore_barrier()
        pltpu.sync_copy(acc.at[pl.ds(s * stripe, stripe)],
                        out_hbm.at[pl.ds(c * n_pad + s * stripe, stripe)])

    return layer_kernel


def _dis_from_degp(degp, n_pad, n):
    deg = degp[:n_pad, 0:1] + degp[n_pad:, 0:1] + 1.0
    rows = lax.broadcasted_iota(jnp.int32, (n_pad, 1), 0)
    return jnp.where(rows < n, lax.rsqrt(deg), 0.0)


def _prep_body(n_pad, n, x_ref, degp_ref, w_ref, out_ref):
    dis = _dis_from_degp(degp_ref[...], n_pad, n)
    h = jnp.dot(x_ref[...], w_ref[...], preferred_element_type=jnp.float32)
    out_ref[...] = h * dis


def _mid_body(n_pad, n, p_ref, h_ref, degp_ref, b_ref, w_ref, out_ref):
    dis = _dis_from_degp(degp_ref[...], n_pad, n)
    p = p_ref[...]
    acc = p[:n_pad] + p[n_pad:] + h_ref[...]
    t = jnp.maximum(acc * dis + b_ref[...], 0.0)
    out_ref[...] = jnp.dot(
        t, w_ref[...], preferred_element_type=jnp.float32) * dis


def _final_body(n_pad, n, p_ref, h_ref, degp_ref, b_ref, wh_ref, bh_ref,
                emb_ref, pred_ref):
    dis = _dis_from_degp(degp_ref[...], n_pad, n)
    p = p_ref[...]
    acc = p[:n_pad] + p[n_pad:] + h_ref[...]
    emb = jnp.maximum(acc * dis + b_ref[...], 0.0)
    pred = jnp.dot(
        emb, wh_ref[...], preferred_element_type=jnp.float32) + bh_ref[...]
    emb_ref[...] = jnp.nan_to_num(emb)
    pred_ref[...] = jnp.nan_to_num(pred)


def kernel(x, edge_index, W1, b1, W2, b2, W3, b3, Wh, bh):
    x = x.astype(jnp.float32)
    n, d = x.shape
    e = edge_index.shape[1]
    n_pad = ((n + NS * CHUNK - 1) // (NS * CHUNK)) * (NS * CHUNK)
    ch = -(-e // (NW * CHUNK))
    e_pad = NW * ch * CHUNK

    fill = jnp.full((e_pad - e,), n, dtype=edge_index.dtype)
    dst = jnp.concatenate([edge_index[1], fill]).reshape(NW, ch, CHUNK)

    tot = -(-e // (NS * CHUNK))
    ch1 = max(1, int(tot * 0.433))
    ch0 = tot - ch1
    e0 = NS * ch0 * CHUNK
    e1_pad = NS * ch1 * CHUNK
    fill1 = jnp.full((e0 + e1_pad - e,), n, dtype=edge_index.dtype)
    src0 = edge_index[0, :e0].reshape(NS, ch0, CHUNK)
    dst0 = edge_index[1, :e0].reshape(NS, ch0, CHUNK)
    src1 = jnp.concatenate(
        [edge_index[0, e0:], fill1]).reshape(NS, ch1, CHUNK)
    dst1 = jnp.concatenate(
        [edge_index[1, e0:], fill1]).reshape(NS, ch1, CHUNK)

    x_pad = jnp.zeros((n_pad, d), jnp.float32).at[:n].set(x)
    z = jnp.zeros((n_pad, d), jnp.float32)
    ones = jnp.ones((CHUNK, d), jnp.float32)
    b1r = b1.reshape(1, d)
    b2r = b2.reshape(1, d)
    b3r = b3.reshape(1, d)
    bhr = bh.reshape(1, 1)

    deg_k = _deg_kernel(n_pad, ch, d)
    lay_k = _layer_kernel(n_pad, ch0, ch1, d)

    prep = pl.pallas_call(
        functools.partial(_prep_body, n_pad, n),
        out_shape=jax.ShapeDtypeStruct((n_pad, d), jnp.float32))
    mid = pl.pallas_call(
        functools.partial(_mid_body, n_pad, n),
        out_shape=jax.ShapeDtypeStruct((n_pad, d), jnp.float32))
    final = pl.pallas_call(
        functools.partial(_final_body, n_pad, n),
        out_shape=(jax.ShapeDtypeStruct((n_pad, d), jnp.float32),
                   jax.ShapeDtypeStruct((n_pad, 1), jnp.float32)))

    degp = deg_k(dst, ones, z)
    h = prep(x_pad, degp, W1)
    p = lay_k(h, src0, dst0, src1, dst1, z)
    h = mid(p, h, degp, b1r, W2)
    p = lay_k(h, src0, dst0, src1, dst1, z)
    h = mid(p, h, degp, b2r, W3)
    p = lay_k(h, src0, dst0, src1, dst1, z)
    emb, pred = final(p, h, degp, b3r, Wh, bhr)
    return emb[:n], pred[:n, 0]
